# baseline (speedup 1.0000x reference)
import functools

import jax
import jax.numpy as jnp
from jax import lax
from jax.experimental import pallas as pl
from jax.experimental.pallas import tpu as pltpu
from jax.experimental.pallas import tpu_sc as plsc

NC, NS, L = 2, 16, 16
NW = NC * NS
RS = 17


def _sc_gather(table, species, mv01, mv23, ability, status1, item, offs,
               chunk, H):
    n_words = table.shape[0]
    bpw = H // NW
    regions = [i * bpw for i in range(6)]

    mesh = plsc.VectorSubcoreMesh(core_axis_name="c", subcore_axis_name="s")

    @functools.partial(
        pl.kernel,
        mesh=mesh,
        out_type=jax.ShapeDtypeStruct((8 * L, H), jnp.float32),
        scratch_types=[
            pltpu.VMEM((n_words,), jnp.float32),
            pltpu.VMEM((6 * bpw,), jnp.int32),
            pltpu.VMEM((8 * L, bpw), jnp.float32),
        ],
        compiler_params=pltpu.CompilerParams(needs_layout_passes=False),
    )
    def body(table_hbm, sp_hbm, m01_hbm, m23_hbm, ab_hbm, st_hbm, it_hbm,
             out_hbm, table_v, idx_v, rows_v):
        wid = lax.axis_index("s") * NC + lax.axis_index("c")
        base = chunk * H + wid * bpw
        pltpu.sync_copy(table_hbm, table_v)
        for reg, src in zip(regions,
                            (sp_hbm, m01_hbm, m23_hbm, ab_hbm, st_hbm, it_hbm)):
            pltpu.sync_copy(src.at[pl.ds(base, bpw)],
                            idx_v.at[pl.ds(reg, bpw)])

        @plsc.parallel_loop(0, bpw // L, unroll=1)
        def _j(j):
            sp = idx_v[pl.ds(regions[0] + j * L, L)] + offs[0]
            p01 = idx_v[pl.ds(regions[1] + j * L, L)]
            p23 = idx_v[pl.ds(regions[2] + j * L, L)]
            ab = idx_v[pl.ds(regions[3] + j * L, L)] + offs[2]
            st = idx_v[pl.ds(regions[4] + j * L, L)] + offs[3]
            it = idx_v[pl.ds(regions[5] + j * L, L)] + offs[4]
            gb = [r * RS for r in (
                sp,
                (p01 & 0xFFFF) + offs[1], lax.shift_right_logical(p01, 16) + offs[1],
                (p23 & 0xFFFF) + offs[1], lax.shift_right_logical(p23, 16) + offs[1],
                ab, st, it)]

            @plsc.parallel_loop(0, L, unroll=8)
            def _c(c):
                for slot in range(8):
                    vals = plsc.load_gather(table_v, [gb[slot] + c])
                    rows_v[slot * L + c, pl.ds(j * L, L)] = vals

        pltpu.sync_copy(rows_v, out_hbm.at[:, pl.ds(wid * bpw, bpw)])

    return body(table, species, mv01, mv23, ability, status1, item)


def _mlp_body(gt_ref, nt_ref, w1a_ref, w1b_ref, b1_ref, w2_ref, b2_ref, o_ref):
    h = lax.dot_general(
        gt_ref[...], w1a_ref[...], (((0,), (0,)), ((), ())),
        preferred_element_type=jnp.float32)
    h = h + lax.dot_general(
        nt_ref[...], w1b_ref[...], (((0,), (0,)), ((), ())),
        preferred_element_type=jnp.float32)
    h = jnp.maximum(h + b1_ref[...], 0.0)
    o_ref[...] = jnp.dot(h, w2_ref[...], preferred_element_type=jnp.float32) + b2_ref[...]


def kernel(species, moves, ability, status1, holdItem, hp, lvl, att, defn, spe,
           spA, spD, pp, exp, species_emb, move_emb, ability_emb, status_emb,
           item_emb, W1, b1, W2, b2):
    B = species.shape[0]
    f32, i32 = jnp.float32, jnp.int32

    o_m = species_emb.shape[0]
    o_a = o_m + move_emb.shape[0]
    o_st = o_a + ability_emb.shape[0]
    o_it = o_st + status_emb.shape[0]
    table = jnp.concatenate(
        [species_emb, move_emb, ability_emb, status_emb, item_emb], axis=0)
    table = jnp.pad(table, ((0, (-table.shape[0]) % 8), (0, RS - L)))

    mv = moves.astype(i32)
    mv01 = mv[:, 0] | (mv[:, 1] << 16)
    mv23 = mv[:, 2] | (mv[:, 3] << 16)

    gt = _sc_gather(
        table.reshape(-1), species.astype(i32), mv01, mv23,
        ability.astype(i32), status1.astype(i32), holdItem.astype(i32),
        (0, o_m, o_a, o_st, o_it), 0, B)

    nt = jnp.concatenate([
        hp[None].astype(f32), lvl[None].astype(f32),
        att[None].astype(f32), defn[None].astype(f32),
        spe[None].astype(f32), spA[None].astype(f32), spD[None].astype(f32),
        jnp.mean(pp.astype(f32), axis=-1)[None], exp[None].astype(f32),
        jnp.zeros((7, B), f32),
    ], axis=0)
    w1a = W1[:128]
    w1b = jnp.concatenate([
        W1[128:129], W1[129:130] / 100.0, W1[130:137],
        jnp.zeros((7, 128), f32),
    ], axis=0)

    BLK = 2048
    out = pl.pallas_call(
        _mlp_body,
        grid=(B // BLK,),
        in_specs=[
            pl.BlockSpec((128, BLK), lambda i: (0, i)),
            pl.BlockSpec((16, BLK), lambda i: (0, i)),
            pl.BlockSpec((128, 128), lambda i: (0, 0)),
            pl.BlockSpec((16, 128), lambda i: (0, 0)),
            pl.BlockSpec((1, 128), lambda i: (0, 0)),
            pl.BlockSpec((128, 128), lambda i: (0, 0)),
            pl.BlockSpec((1, 128), lambda i: (0, 0)),
        ],
        out_specs=pl.BlockSpec((BLK, 128), lambda i: (i, 0)),
        out_shape=jax.ShapeDtypeStruct((B, 128), f32),
    )(gt, nt, w1a, w1b, b1.reshape(1, 128), W2, b2.reshape(1, 128))
    return out


# speedup vs baseline: 1.0600x; 1.0600x over previous
import functools

import jax
import jax.numpy as jnp
from jax import lax
from jax.experimental import pallas as pl
from jax.experimental.pallas import tpu as pltpu
from jax.experimental.pallas import tpu_sc as plsc

NC, NS, L = 2, 16, 16
NW = NC * NS
RS = 17


def _sc_gather(table, species, mv01, mv23, ability, status1, item, offs,
               chunk, H):
    n_words = table.shape[0]
    bpw = H // NW
    regions = [i * bpw for i in range(6)]

    mesh = plsc.VectorSubcoreMesh(core_axis_name="c", subcore_axis_name="s")

    @functools.partial(
        pl.kernel,
        mesh=mesh,
        out_type=jax.ShapeDtypeStruct((8 * L, H), jnp.float32),
        scratch_types=[
            pltpu.VMEM((n_words,), jnp.float32),
            pltpu.VMEM((6 * bpw,), jnp.int32),
            pltpu.VMEM((8 * L, bpw), jnp.float32),
        ],
        compiler_params=pltpu.CompilerParams(needs_layout_passes=False),
    )
    def body(table_hbm, sp_hbm, m01_hbm, m23_hbm, ab_hbm, st_hbm, it_hbm,
             out_hbm, table_v, idx_v, rows_v):
        wid = lax.axis_index("s") * NC + lax.axis_index("c")
        base = chunk * H + wid * bpw
        pltpu.sync_copy(table_hbm, table_v)
        for reg, src in zip(regions,
                            (sp_hbm, m01_hbm, m23_hbm, ab_hbm, st_hbm, it_hbm)):
            pltpu.sync_copy(src.at[pl.ds(base, bpw)],
                            idx_v.at[pl.ds(reg, bpw)])

        @plsc.parallel_loop(0, bpw // L, unroll=1)
        def _j(j):
            sp = idx_v[pl.ds(regions[0] + j * L, L)] + offs[0]
            p01 = idx_v[pl.ds(regions[1] + j * L, L)]
            p23 = idx_v[pl.ds(regions[2] + j * L, L)]
            ab = idx_v[pl.ds(regions[3] + j * L, L)] + offs[2]
            st = idx_v[pl.ds(regions[4] + j * L, L)] + offs[3]
            it = idx_v[pl.ds(regions[5] + j * L, L)] + offs[4]
            gb = [r * RS for r in (
                sp,
                (p01 & 0xFFFF) + offs[1], lax.shift_right_logical(p01, 16) + offs[1],
                (p23 & 0xFFFF) + offs[1], lax.shift_right_logical(p23, 16) + offs[1],
                ab, st, it)]

            @plsc.parallel_loop(0, L, unroll=4)
            def _c(c):
                for slot in range(8):
                    vals = plsc.load_gather(table_v, [gb[slot] + c])
                    rows_v[slot * L + c, pl.ds(j * L, L)] = vals

        pltpu.sync_copy(rows_v, out_hbm.at[:, pl.ds(wid * bpw, bpw)])

    return body(table, species, mv01, mv23, ability, status1, item)


def _mlp_body(gt_ref, nt_ref, w1a_ref, w1b_ref, b1_ref, w2_ref, b2_ref, o_ref):
    h = lax.dot_general(
        gt_ref[...], w1a_ref[...], (((0,), (0,)), ((), ())),
        preferred_element_type=jnp.float32)
    h = h + lax.dot_general(
        nt_ref[...], w1b_ref[...], (((0,), (0,)), ((), ())),
        preferred_element_type=jnp.float32)
    h = jnp.maximum(h + b1_ref[...], 0.0)
    o_ref[...] = jnp.dot(h, w2_ref[...], preferred_element_type=jnp.float32) + b2_ref[...]


def kernel(species, moves, ability, status1, holdItem, hp, lvl, att, defn, spe,
           spA, spD, pp, exp, species_emb, move_emb, ability_emb, status_emb,
           item_emb, W1, b1, W2, b2):
    B = species.shape[0]
    f32, i32 = jnp.float32, jnp.int32

    o_m = species_emb.shape[0]
    o_a = o_m + move_emb.shape[0]
    o_st = o_a + ability_emb.shape[0]
    o_it = o_st + status_emb.shape[0]
    table = jnp.concatenate(
        [species_emb, move_emb, ability_emb, status_emb, item_emb], axis=0)
    table = jnp.pad(table, ((0, (-table.shape[0]) % 8), (0, RS - L)))

    mv = moves.astype(i32)
    mv01 = mv[:, 0] | (mv[:, 1] << 16)
    mv23 = mv[:, 2] | (mv[:, 3] << 16)

    gt = _sc_gather(
        table.reshape(-1), species.astype(i32), mv01, mv23,
        ability.astype(i32), status1.astype(i32), holdItem.astype(i32),
        (0, o_m, o_a, o_st, o_it), 0, B)

    nt = jnp.concatenate([
        hp[None].astype(f32), lvl[None].astype(f32),
        att[None].astype(f32), defn[None].astype(f32),
        spe[None].astype(f32), spA[None].astype(f32), spD[None].astype(f32),
        jnp.mean(pp.astype(f32), axis=-1)[None], exp[None].astype(f32),
        jnp.zeros((7, B), f32),
    ], axis=0)
    w1a = W1[:128]
    w1b = jnp.concatenate([
        W1[128:129], W1[129:130] / 100.0, W1[130:137],
        jnp.zeros((7, 128), f32),
    ], axis=0)

    BLK = 4096
    out = pl.pallas_call(
        _mlp_body,
        grid=(B // BLK,),
        in_specs=[
            pl.BlockSpec((128, BLK), lambda i: (0, i)),
            pl.BlockSpec((16, BLK), lambda i: (0, i)),
            pl.BlockSpec((128, 128), lambda i: (0, 0)),
            pl.BlockSpec((16, 128), lambda i: (0, 0)),
            pl.BlockSpec((1, 128), lambda i: (0, 0)),
            pl.BlockSpec((128, 128), lambda i: (0, 0)),
            pl.BlockSpec((1, 128), lambda i: (0, 0)),
        ],
        out_specs=pl.BlockSpec((BLK, 128), lambda i: (i, 0)),
        out_shape=jax.ShapeDtypeStruct((B, 128), f32),
    )(gt, nt, w1a, w1b, b1.reshape(1, 128), W2, b2.reshape(1, 128))
    return out


# speedup vs baseline: 1.0710x; 1.0103x over previous
import functools

import jax
import jax.numpy as jnp
from jax import lax
from jax.experimental import pallas as pl
from jax.experimental.pallas import tpu as pltpu
from jax.experimental.pallas import tpu_sc as plsc

NC, NS, L = 2, 16, 16
NW = NC * NS
RS = 17


def _sc_gather(table, species, mv01, mv23, ability, status1, item, offs,
               chunk, H):
    n_words = table.shape[0]
    bpw = H // NW
    regions = [i * bpw for i in range(6)]

    mesh = plsc.VectorSubcoreMesh(core_axis_name="c", subcore_axis_name="s")

    @functools.partial(
        pl.kernel,
        mesh=mesh,
        out_type=jax.ShapeDtypeStruct((8 * L, H), jnp.float32),
        scratch_types=[
            pltpu.VMEM((n_words,), jnp.float32),
            pltpu.VMEM((6 * bpw,), jnp.int32),
            pltpu.VMEM((8 * L, bpw), jnp.float32),
        ],
        compiler_params=pltpu.CompilerParams(needs_layout_passes=False),
    )
    def body(table_hbm, sp_hbm, m01_hbm, m23_hbm, ab_hbm, st_hbm, it_hbm,
             out_hbm, table_v, idx_v, rows_v):
        wid = lax.axis_index("s") * NC + lax.axis_index("c")
        base = chunk * H + wid * bpw
        pltpu.sync_copy(table_hbm, table_v)
        for reg, src in zip(regions,
                            (sp_hbm, m01_hbm, m23_hbm, ab_hbm, st_hbm, it_hbm)):
            pltpu.sync_copy(src.at[pl.ds(base, bpw)],
                            idx_v.at[pl.ds(reg, bpw)])

        @plsc.parallel_loop(0, bpw // L, unroll=1)
        def _j(j):
            sp = idx_v[pl.ds(regions[0] + j * L, L)] + offs[0]
            p01 = idx_v[pl.ds(regions[1] + j * L, L)]
            p23 = idx_v[pl.ds(regions[2] + j * L, L)]
            ab = idx_v[pl.ds(regions[3] + j * L, L)] + offs[2]
            st = idx_v[pl.ds(regions[4] + j * L, L)] + offs[3]
            it = idx_v[pl.ds(regions[5] + j * L, L)] + offs[4]
            gb = [r * RS for r in (
                sp,
                (p01 & 0xFFFF) + offs[1], lax.shift_right_logical(p01, 16) + offs[1],
                (p23 & 0xFFFF) + offs[1], lax.shift_right_logical(p23, 16) + offs[1],
                ab, st, it)]

            @plsc.parallel_loop(0, L, unroll=4)
            def _c(c):
                for slot in range(8):
                    vals = plsc.load_gather(table_v, [gb[slot] + c])
                    rows_v[slot * L + c, pl.ds(j * L, L)] = vals

        pltpu.sync_copy(rows_v, out_hbm.at[:, pl.ds(wid * bpw, bpw)])

    return body(table, species, mv01, mv23, ability, status1, item)


def _mlp_body(gt_ref, nt_ref, w1a_ref, w1b_ref, b1_ref, w2_ref, b2_ref, o_ref):
    h = lax.dot_general(
        gt_ref[...], w1a_ref[...], (((0,), (0,)), ((), ())),
        preferred_element_type=jnp.float32)
    h = h + lax.dot_general(
        nt_ref[...], w1b_ref[...], (((0,), (0,)), ((), ())),
        preferred_element_type=jnp.float32)
    h = jnp.maximum(h + b1_ref[...], 0.0)
    o_ref[...] = jnp.dot(h, w2_ref[...], preferred_element_type=jnp.float32) + b2_ref[...]


def kernel(species, moves, ability, status1, holdItem, hp, lvl, att, defn, spe,
           spA, spD, pp, exp, species_emb, move_emb, ability_emb, status_emb,
           item_emb, W1, b1, W2, b2):
    B = species.shape[0]
    f32, i32 = jnp.float32, jnp.int32

    o_m = species_emb.shape[0]
    o_a = o_m + move_emb.shape[0]
    o_st = o_a + ability_emb.shape[0]
    o_it = o_st + status_emb.shape[0]
    table = jnp.concatenate(
        [species_emb, move_emb, ability_emb, status_emb, item_emb], axis=0)
    table = jnp.pad(table, ((0, (-table.shape[0]) % 8), (0, RS - L)))

    mv = moves.astype(i32)
    mv01 = mv[:, 0] | (mv[:, 1] << 16)
    mv23 = mv[:, 2] | (mv[:, 3] << 16)

    gt = _sc_gather(
        table.reshape(-1), species.astype(i32), mv01, mv23,
        ability.astype(i32), status1.astype(i32), holdItem.astype(i32),
        (0, o_m, o_a, o_st, o_it), 0, B)

    nt = jnp.concatenate([
        hp[None].astype(f32), lvl[None].astype(f32),
        att[None].astype(f32), defn[None].astype(f32),
        spe[None].astype(f32), spA[None].astype(f32), spD[None].astype(f32),
        jnp.mean(pp.astype(f32), axis=-1)[None], exp[None].astype(f32),
        jnp.zeros((7, B), f32),
    ], axis=0)
    w1a = W1[:128]
    w1b = jnp.concatenate([
        W1[128:129], W1[129:130] / 100.0, W1[130:137],
        jnp.zeros((7, 128), f32),
    ], axis=0)

    BLK = 8192
    out = pl.pallas_call(
        _mlp_body,
        grid=(B // BLK,),
        in_specs=[
            pl.BlockSpec((128, BLK), lambda i: (0, i)),
            pl.BlockSpec((16, BLK), lambda i: (0, i)),
            pl.BlockSpec((128, 128), lambda i: (0, 0)),
            pl.BlockSpec((16, 128), lambda i: (0, 0)),
            pl.BlockSpec((1, 128), lambda i: (0, 0)),
            pl.BlockSpec((128, 128), lambda i: (0, 0)),
            pl.BlockSpec((1, 128), lambda i: (0, 0)),
        ],
        out_specs=pl.BlockSpec((BLK, 128), lambda i: (i, 0)),
        out_shape=jax.ShapeDtypeStruct((B, 128), f32),
    )(gt, nt, w1a, w1b, b1.reshape(1, 128), W2, b2.reshape(1, 128))
    return out
